# Initial kernel scaffold; baseline (speedup 1.0000x reference)
import jax
import jax.numpy as jnp
from jax.experimental import pallas as pl


def kernel(input, target):
    raise NotImplementedError("write your pallas kernel here")


# speedup vs baseline: 155.1882x; 155.1882x over previous
import functools

import jax
import jax.numpy as jnp
from jax.experimental import pallas as pl

_N_CLASSES = 19


def _body(x_ref, t_ref, acc_ref):
    @pl.when((pl.program_id(0) == 0) & (pl.program_id(1) == 0))
    def _init():
        acc_ref[...] = jnp.zeros_like(acc_ref)

    t = t_ref[0]
    c_dim = x_ref.shape[1]

    m = x_ref[0, 0]
    am = jnp.zeros_like(t)
    for c in range(1, c_dim):
        xc = x_ref[0, c]
        hit = xc > m
        m = jnp.where(hit, xc, m)
        am = jnp.where(hit, c, am)

    s = jnp.zeros_like(m)
    g = jnp.zeros_like(m)
    for c in range(c_dim):
        xc = x_ref[0, c]
        s = s + jnp.exp(xc - m)
        g = g + jnp.where(t == c, xc, 0.0)

    ce = m + jnp.log(s) - g
    fn = am != t

    for c in range(c_dim):
        eq = t == c
        acc_ref[0, c] += jnp.sum(eq.astype(jnp.float32), axis=0)
        acc_ref[1, c] += jnp.sum((eq & fn).astype(jnp.float32), axis=0)
        acc_ref[2, c] += jnp.sum(jnp.where(eq, ce, 0.0), axis=0)


@functools.partial(jax.jit, static_argnames=("interpret",))
def kernel(input, target, interpret=False):
    b_dim, c_dim, h_dim, w_dim = input.shape
    bh = 64
    acc = pl.pallas_call(
        _body,
        grid=(b_dim, h_dim // bh),
        in_specs=[
            pl.BlockSpec((1, c_dim, bh, w_dim), lambda b, h: (b, 0, h, 0)),
            pl.BlockSpec((1, bh, w_dim), lambda b, h: (b, h, 0)),
        ],
        out_specs=pl.BlockSpec((3, c_dim, w_dim), lambda b, h: (0, 0, 0)),
        out_shape=jax.ShapeDtypeStruct((3, c_dim, w_dim), jnp.float32),
        interpret=interpret,
    )(input, target)
    gt = jnp.sum(acc[0], axis=-1)
    fn = jnp.sum(acc[1], axis=-1)
    ces = jnp.sum(acc[2], axis=-1)
    weight = jnp.where(fn > 0, fn, 1.0) / jnp.where(gt > 0, gt, 1.0)
    return jnp.sum(weight * ces) / (b_dim * h_dim * w_dim)


# speedup vs baseline: 218.7972x; 1.4099x over previous
import functools

import jax
import jax.numpy as jnp
from jax.experimental import pallas as pl

_N_CLASSES = 19


def _body(x_ref, t_ref, acc_ref):
    @pl.when((pl.program_id(0) == 0) & (pl.program_id(1) == 0))
    def _init():
        acc_ref[...] = jnp.zeros_like(acc_ref)

    t = t_ref[0]
    c_dim = x_ref.shape[1]
    bh = t.shape[0]
    ones = jnp.ones((1, bh), dtype=jnp.float32)
    dot = functools.partial(
        jax.lax.dot_general,
        dimension_numbers=(((1,), (0,)), ((), ())),
        preferred_element_type=jnp.float32,
    )

    m = x_ref[0, 0]
    for c in range(1, c_dim):
        m = jnp.maximum(m, x_ref[0, c])

    s = jnp.zeros_like(m)
    for c in range(c_dim):
        xc = x_ref[0, c]
        s = s + jnp.exp(xc - m)
        eqf = (t == c).astype(jnp.float32)
        fnv = eqf * (xc < m).astype(jnp.float32)
        gxv = eqf * xc
        acc_ref[0, c] += dot(ones, eqf)[0]
        acc_ref[1, c] += dot(ones, fnv)[0]
        acc_ref[2, c] += dot(ones, gxv)[0]

    lse = m + jnp.log(s)
    for c in range(c_dim):
        eqf = (t == c).astype(jnp.float32)
        acc_ref[3, c] += dot(ones, eqf * lse)[0]


@functools.partial(jax.jit, static_argnames=("interpret",))
def kernel(input, target, interpret=False):
    b_dim, c_dim, h_dim, w_dim = input.shape
    bh = 64
    acc = pl.pallas_call(
        _body,
        grid=(b_dim, h_dim // bh),
        in_specs=[
            pl.BlockSpec((1, c_dim, bh, w_dim), lambda b, h: (b, 0, h, 0)),
            pl.BlockSpec((1, bh, w_dim), lambda b, h: (b, h, 0)),
        ],
        out_specs=pl.BlockSpec((4, c_dim, w_dim), lambda b, h: (0, 0, 0)),
        out_shape=jax.ShapeDtypeStruct((4, c_dim, w_dim), jnp.float32),
        interpret=interpret,
    )(input, target)
    gt = jnp.sum(acc[0], axis=-1)
    fn = jnp.sum(acc[1], axis=-1)
    gx = jnp.sum(acc[2], axis=-1)
    lse = jnp.sum(acc[3], axis=-1)
    ces = lse - gx
    weight = jnp.where(fn > 0, fn, 1.0) / jnp.where(gt > 0, gt, 1.0)
    return jnp.sum(weight * ces) / (b_dim * h_dim * w_dim)
